# Initial kernel scaffold; baseline (speedup 1.0000x reference)
import jax
import jax.numpy as jnp
from jax.experimental import pallas as pl


def kernel(embeddings, labels, W1, b1, W2, b2, queue_embeddings, queue_labels):
    raise NotImplementedError("write your pallas kernel here")


# speedup vs baseline: 52.0108x; 52.0108x over previous
import functools

import jax
import jax.numpy as jnp
from jax import lax
from jax.experimental import pallas as pl
from jax.experimental.pallas import tpu as pltpu

_N = 256
_EMBED_DIM = 768
_PROJ_DIM = 128
_QUEUE_SIZE = 65536
_TEMPERATURE = 0.07
_HARD_K = 256

_TILE = 2048
_NTILES = _QUEUE_SIZE // _TILE
_NPASS = 14
_INV_T = 1.0 / _TEMPERATURE
_SENTINEL = -2.0
_LO0 = -1.02
_HI0 = 1.02

_HIGH = lax.Precision.HIGHEST


def _rowsum(x):
    return jnp.sum(x, axis=1, keepdims=True)


def _body(emb_ref, lab_col_ref, lab_row_ref, w1_ref, b1_ref, w2_ref, b2_ref,
          qe_ref, ql_ref, out_ref,
          z_s, qs_s, nopp_s, fullsum_s, fb_s, lo_s, hi_s, chi_s):
    i = pl.program_id(0)

    @pl.when(i == 0)
    def _init():
        emb = emb_ref[...]
        h = lax.dot_general(emb, w1_ref[...], (((1,), (0,)), ((), ())),
                            precision=_HIGH, preferred_element_type=jnp.float32)
        h = jnp.maximum(h + b1_ref[...], 0.0)
        zp = lax.dot_general(h, w2_ref[...], (((1,), (0,)), ((), ())),
                             precision=_HIGH, preferred_element_type=jnp.float32)
        zp = zp + b2_ref[...]
        n = jnp.sqrt(_rowsum(zp * zp))
        z_s[...] = zp / jnp.maximum(n, 1e-12)
        nopp_s[...] = jnp.zeros_like(nopp_s)
        fullsum_s[...] = jnp.zeros_like(fullsum_s)
        lo_s[...] = jnp.full_like(lo_s, _LO0)
        hi_s[...] = jnp.full_like(hi_s, _HI0)
        chi_s[...] = jnp.zeros_like(chi_s)

    @pl.when(i < _NTILES)
    def _tile():
        z = z_s[...]
        qt = qe_ref[...]
        qs = lax.dot_general(z.astype(jnp.bfloat16), qt.astype(jnp.bfloat16),
                             (((1,), (1,)), ((), ())),
                             preferred_element_type=jnp.float32)
        ql = ql_ref[0]
        lab = lab_col_ref[...]
        opp = (ql != lab) & (ql >= 0)
        e = jnp.exp(qs * _INV_T)
        nopp_s[...] += _rowsum(jnp.where(opp, 1.0, 0.0))
        fullsum_s[...] += _rowsum(jnp.where(opp, e, 0.0))
        qs_s[i] = jnp.where(opp, qs, _SENTINEL).astype(jnp.bfloat16)

        @pl.when(i == 0)
        def _fb():
            fb_s[...] = _rowsum(e[:, :_HARD_K])

    @pl.when(i == _NTILES)
    def _finish():
        def _pass(_, carry):
            lo = lo_s[...]
            hi = hi_s[...]
            mid = 0.5 * (lo + hi)
            cnt = jnp.zeros_like(lo)
            for c in range(_NTILES):
                v = qs_s[c].astype(jnp.float32)
                cnt += _rowsum(jnp.where(v >= mid, 1.0, 0.0))
            ge = cnt >= float(_HARD_K)
            lo_s[...] = jnp.where(ge, mid, lo)
            hi_s[...] = jnp.where(ge, hi, mid)
            chi_s[...] = jnp.where(ge, chi_s[...], cnt)
            return carry

        lax.fori_loop(0, _NPASS, _pass, 0)

        lo = lo_s[...]
        hi = hi_s[...]
        chi = chi_s[...]
        s_above = jnp.zeros_like(lo)
        for c in range(_NTILES):
            v = qs_s[c].astype(jnp.float32)
            s_above += _rowsum(jnp.where(v >= hi, jnp.exp(v * _INV_T), 0.0))
        mid = 0.5 * (lo + hi)
        s_hard = s_above + jnp.maximum(float(_HARD_K) - chi, 0.0) * jnp.exp(mid * _INV_T)

        nopp = nopp_s[...]
        s_hard = jnp.where(nopp <= float(_HARD_K), fullsum_s[...], s_hard)
        s_hard = jnp.where(nopp == 0.0, fb_s[...], s_hard)

        z = z_s[...]
        sb = lax.dot_general(z, z, (((1,), (1,)), ((), ())),
                             precision=_HIGH, preferred_element_type=jnp.float32)
        sb = sb * _INV_T
        lab_c = lab_col_ref[...]
        lab_r = lab_row_ref[...]
        same = lab_c == lab_r
        rr = lax.broadcasted_iota(jnp.int32, (_N, _N), 0)
        cc = lax.broadcasted_iota(jnp.int32, (_N, _N), 1)
        eye = rr == cc
        pos = same & (~eye)
        neg = ~same

        esb = jnp.exp(sb)
        e_neg = _rowsum(jnp.where(neg, esb, 0.0)) + s_hard
        terms = jnp.where(pos, jnp.log(esb + e_neg) - sb, 0.0)
        total = jnp.sum(terms, axis=(0, 1), keepdims=True)
        cnt_pos = jnp.sum(jnp.where(pos, 1.0, 0.0), axis=(0, 1), keepdims=True)
        out_ref[...] = jnp.where(cnt_pos > 0.0,
                                 total / jnp.maximum(cnt_pos, 1.0),
                                 jnp.zeros_like(total))


@jax.jit
def kernel(embeddings, labels, W1, b1, W2, b2, queue_embeddings, queue_labels):
    lab_col = labels.reshape(_N, 1)
    lab_row = labels.reshape(1, _N)
    b1r = b1.reshape(1, _EMBED_DIM)
    b2r = b2.reshape(1, _PROJ_DIM)
    ql3 = queue_labels.reshape(_NTILES, 1, _TILE)

    grid = (_NTILES + 1,)
    zero = lambda i: (0, 0)
    tile_idx = lambda i: (jnp.minimum(i, _NTILES - 1), 0)
    tile_idx3 = lambda i: (jnp.minimum(i, _NTILES - 1), 0, 0)

    out = pl.pallas_call(
        _body,
        grid=grid,
        in_specs=[
            pl.BlockSpec((_N, _EMBED_DIM), zero),
            pl.BlockSpec((_N, 1), zero),
            pl.BlockSpec((1, _N), zero),
            pl.BlockSpec((_EMBED_DIM, _EMBED_DIM), zero),
            pl.BlockSpec((1, _EMBED_DIM), zero),
            pl.BlockSpec((_EMBED_DIM, _PROJ_DIM), zero),
            pl.BlockSpec((1, _PROJ_DIM), zero),
            pl.BlockSpec((_TILE, _PROJ_DIM), tile_idx),
            pl.BlockSpec((1, 1, _TILE), tile_idx3),
        ],
        out_specs=pl.BlockSpec((1, 1), zero),
        out_shape=jax.ShapeDtypeStruct((1, 1), jnp.float32),
        scratch_shapes=[
            pltpu.VMEM((_N, _PROJ_DIM), jnp.float32),
            pltpu.VMEM((_NTILES, _N, _TILE), jnp.bfloat16),
            pltpu.VMEM((_N, 1), jnp.float32),
            pltpu.VMEM((_N, 1), jnp.float32),
            pltpu.VMEM((_N, 1), jnp.float32),
            pltpu.VMEM((_N, 1), jnp.float32),
            pltpu.VMEM((_N, 1), jnp.float32),
            pltpu.VMEM((_N, 1), jnp.float32),
        ],
    )(embeddings, lab_col, lab_row, W1, b1r, W2, b2r,
      queue_embeddings, ql3)
    return out[0, 0]
